# baseline (speedup 1.0000x reference)
import jax
import jax.numpy as jnp
from jax.experimental import pallas as pl
from jax.experimental.pallas import tpu as pltpu

_NUM_MODULES = 4
_NB = 256
_NBF = 512


def _aug(p):
    r = p.shape[0]
    ones = jnp.ones((r, 1), jnp.float32)
    zeros = jnp.zeros((r, 11), jnp.float32)
    p2 = jnp.sum(p * p, axis=1, keepdims=True)
    return jnp.concatenate([-2.0 * p, ones, zeros, p2], axis=1)


def _bdot(a, b):
    bf = jnp.bfloat16
    return jnp.dot(a.astype(bf), b.astype(bf),
                   preferred_element_type=jnp.float32)


def _feat_kernel(x_ref, seed_ref, w1_ref, b1_ref, w2_ref, b2_ref, w3_ref,
                 b3_ref, qp_ref, vv_ref):
    x = x_ref[0] - seed_ref[0]
    h = jnp.maximum(_bdot(x, w1_ref[...]) + b1_ref[...], 0.0)
    h = jnp.maximum(_bdot(h, w2_ref[...]) + b2_ref[...], 0.0)
    o = _bdot(h, w3_ref[...]) + b3_ref[...]
    q = x
    for i in range(_NUM_MODULES):
        q2 = jnp.sum(q * q, axis=1, keepdims=True)
        qp_ref[i, 0] = jnp.concatenate([q, q2], axis=1)
        q = q + o[:, 3 * i:3 * i + 3]
        vv_ref[i, 0] = _aug(q)


def _target_kernel(ct_ref, st_ref, n0_ref, n1_ref, sa_ref, sb_ref, rt_ref):
    ct = ct_ref[0] - st_ref[0]
    m = ct.shape[1]
    t0 = ct + n0_ref[0] * sa_ref[0]
    t1 = ct + n1_ref[0] * sb_ref[0]
    pad = jnp.zeros((11, m), jnp.float32)
    one = jnp.ones((1, m), jnp.float32)
    for i, t in enumerate((t0, t1, ct, ct)):
        r2 = jnp.sum(t * t, axis=0, keepdims=True)
        rt_ref[i, 0] = jnp.concatenate([t, r2, pad, one], axis=0)


def _knn_kernel(qp_ref, vv_ref, rt_ref, out_ref):
    rt = rt_ref[0, 0]
    qp = qp_ref[0, 0]
    m = rt.shape[1]
    qr = _bdot(qp[:, 0:3], rt[0:3, :])
    s = (qp[:, 3:4] - 2.0 * qr) + rt[3:4, :]
    d = jnp.dot(vv_ref[0, 0], rt,
                precision=jax.lax.Precision.HIGHEST)
    iota = jax.lax.broadcasted_iota(jnp.int32, s.shape, 1)
    rmin = jnp.min(s, axis=1, keepdims=True)
    mi = jnp.where(s == rmin, iota, m)
    idx = jnp.min(mi, axis=1, keepdims=True)
    part = jnp.sum(jnp.where(mi == idx, d, 0.0)).reshape(1, 1)

    @pl.when(pl.program_id(2) == 0)
    def _init():
        out_ref[0, 0] = part

    @pl.when(pl.program_id(2) != 0)
    def _acc():
        out_ref[0, 0] += part


def kernel(pcl_noisy, pcl_clean, pcl_seeds, pcl_std, W1, b1, W2, b2, W3, b3):
    B, N, _ = pcl_noisy.shape
    M = pcl_clean.shape[1]
    f32 = jnp.float32

    noise_key = jax.random.key(42)
    n0 = jax.random.normal(jax.random.fold_in(noise_key, 0), (B, M, 3), f32)
    n1 = jax.random.normal(jax.random.fold_in(noise_key, 1), (B, M, 3), f32)

    qp, vv = pl.pallas_call(
        _feat_kernel,
        grid=(B, N // _NBF),
        in_specs=[
            pl.BlockSpec((1, _NBF, 3), lambda b, n: (b, n, 0)),
            pl.BlockSpec((1, 1, 3), lambda b, n: (b, 0, 0)),
            pl.BlockSpec((3, 128), lambda b, n: (0, 0)),
            pl.BlockSpec((1, 128), lambda b, n: (0, 0)),
            pl.BlockSpec((128, 128), lambda b, n: (0, 0)),
            pl.BlockSpec((1, 128), lambda b, n: (0, 0)),
            pl.BlockSpec((128, 12), lambda b, n: (0, 0)),
            pl.BlockSpec((1, 12), lambda b, n: (0, 0)),
        ],
        out_specs=[
            pl.BlockSpec((_NUM_MODULES, 1, _NBF, 4), lambda b, n: (0, b, n, 0)),
            pl.BlockSpec((_NUM_MODULES, 1, _NBF, 16), lambda b, n: (0, b, n, 0)),
        ],
        out_shape=[
            jax.ShapeDtypeStruct((_NUM_MODULES, B, N, 4), f32),
            jax.ShapeDtypeStruct((_NUM_MODULES, B, N, 16), f32),
        ],
        compiler_params=pltpu.CompilerParams(
            dimension_semantics=("parallel", "parallel")),
    )(pcl_noisy, pcl_seeds, W1, b1.reshape(1, -1), W2, b2.reshape(1, -1),
      W3, b3.reshape(1, -1))

    ct = jnp.transpose(pcl_clean, (0, 2, 1))
    st = jnp.transpose(pcl_seeds, (0, 2, 1))
    n0t = jnp.transpose(n0, (0, 2, 1))
    n1t = jnp.transpose(n1, (0, 2, 1))
    sa = (pcl_std / 4.0).reshape(B, 1, 1)
    sb = (pcl_std / 16.0).reshape(B, 1, 1)
    rt = pl.pallas_call(
        _target_kernel,
        grid=(B,),
        in_specs=[
            pl.BlockSpec((1, 3, M), lambda b: (b, 0, 0)),
            pl.BlockSpec((1, 3, 1), lambda b: (b, 0, 0)),
            pl.BlockSpec((1, 3, M), lambda b: (b, 0, 0)),
            pl.BlockSpec((1, 3, M), lambda b: (b, 0, 0)),
            pl.BlockSpec((1, 1, 1), lambda b: (b, 0, 0)),
            pl.BlockSpec((1, 1, 1), lambda b: (b, 0, 0)),
        ],
        out_specs=pl.BlockSpec((_NUM_MODULES, 1, 16, M), lambda b: (0, b, 0, 0)),
        out_shape=jax.ShapeDtypeStruct((_NUM_MODULES, B, 16, M), f32),
        compiler_params=pltpu.CompilerParams(
            dimension_semantics=("parallel",)),
    )(ct, st, n0t, n1t, sa, sb)

    loss4 = pl.pallas_call(
        _knn_kernel,
        grid=(_NUM_MODULES, B, N // _NB),
        in_specs=[
            pl.BlockSpec((1, 1, _NB, 4), lambda i, b, n: (i, b, n, 0)),
            pl.BlockSpec((1, 1, _NB, 16), lambda i, b, n: (i, b, n, 0)),
            pl.BlockSpec((1, 1, 16, M), lambda i, b, n: (i, b, 0, 0)),
        ],
        out_specs=pl.BlockSpec((1, 1, 1, 1), lambda i, b, n: (i, b, 0, 0)),
        out_shape=jax.ShapeDtypeStruct((_NUM_MODULES, B, 1, 1), f32),
        compiler_params=pltpu.CompilerParams(
            dimension_semantics=("parallel", "parallel", "arbitrary")),
    )(qp, vv, rt)

    return jnp.sum(loss4[:, :, 0, 0]) / B
